# baseline (speedup 1.0000x reference)
import functools

import jax
import jax.numpy as jnp
from jax import lax
from jax.experimental import pallas as pl
from jax.experimental.pallas import tpu as pltpu
import jax.experimental.pallas.tpu_sc as plsc

N = 10000
E = 320000
D_IN = 128
D_HID = 128
N_CLS = 40

NC, NS = 2, 16
NW = NC * NS
CR = 125
TCH = E // CR
WCH = TCH // NW
NPD = 10240
RPT = NPD // NS
DC = 48

_MESH = dict(core_axis_name="c", subcore_axis_name="s", num_cores=NC,
             num_subcores=NS)


_DL = 500
_DNS = E // (NW * _DL)


def _deg_kernel(dst_hbm, degp_hbm, dst_v, ones_v, zeros_v, deg_sh, sem):
  c = lax.axis_index("c")
  s = lax.axis_index("s")
  w = s * NC + c

  z16 = jnp.zeros((16,), jnp.float32)
  for i in range(512 // 16):
    ones_v[pl.ds(i * 16, 16)] = z16 + 1.0
  for i in range(128 // 16):
    zeros_v[pl.ds(i * 16, 16)] = z16
  for i in range(RPT // 128):
    pltpu.sync_copy(zeros_v, deg_sh.at[pl.ds(s * RPT + i * 128, 128)])
  plsc.subcore_barrier()

  pltpu.sync_copy(dst_hbm.at[pl.ds(w * _DNS, _DNS)], dst_v)
  def body(j, _):
    pltpu.sync_copy(ones_v.at[pl.ds(0, _DL)], deg_sh.at[dst_v.at[j]],
                    add=True)
    return 0
  lax.fori_loop(0, _DNS, body, 0)
  plsc.subcore_barrier()

  pltpu.sync_copy(deg_sh.at[pl.ds(s * RPT, RPT)],
                  degp_hbm.at[c, pl.ds(s * RPT, RPT)])


def _make_agg(d, multi, n_half):
  L = multi * CR
  ns = WCH // (n_half * multi)
  assert ns % 2 == 0

  rows_t = pltpu.VMEM((2, L, d), jnp.float32)

  @functools.partial(
      pl.kernel,
      out_type=jax.ShapeDtypeStruct((NC, NPD, d), jnp.float32),
      mesh=plsc.VectorSubcoreMesh(**_MESH),
      compiler_params=pltpu.CompilerParams(use_tc_tiling_on_sc=(d % 128 == 0)),
      scratch_types=[
          pltpu.VMEM((ns, L), jnp.int32),
          pltpu.VMEM((ns, L), jnp.int32),
          rows_t,
          pltpu.VMEM_SHARED((NPD, d), jnp.float32),
          pltpu.SemaphoreType.DMA,
          pltpu.SemaphoreType.DMA,
      ],
  )
  def agg(h_hbm, src_hbm, dst_hbm, out_hbm,
          src_v, dst_v, rows_v, acc_sh, gsem, ssem):
    c = lax.axis_index("c")
    s = lax.axis_index("s")
    w = s * NC + c

    @pl.when(c == 0)
    def _():
      pltpu.sync_copy(h_hbm.at[pl.ds(s * RPT, RPT)],
                      acc_sh.at[pl.ds(s * RPT, RPT)])

    @pl.when(c == 1)
    def _():
      z16 = jnp.zeros((16,), jnp.float32)
      def zb(i, _):
        for k in range(d // 16):
          rows_v[0, i, pl.ds(k * 16, 16)] = z16
        return 0
      lax.fori_loop(0, CR, zb, 0)
      for i in range(5):
        pltpu.sync_copy(rows_v.at[0, pl.ds(0, 120)],
                        acc_sh.at[pl.ds(s * RPT + i * 120, 120)])
      pltpu.sync_copy(rows_v.at[0, pl.ds(0, 40)],
                      acc_sh.at[pl.ds(s * RPT + 600, 40)])
    plsc.subcore_barrier()

    def idx(v, t):
      return v.at[t]

    for k in range(n_half):
      pltpu.sync_copy(src_hbm.at[pl.ds((w * n_half + k) * ns, ns)], src_v)
      pltpu.sync_copy(dst_hbm.at[pl.ds((w * n_half + k) * ns, ns)], dst_v)

      def gather(t, b):
        pltpu.async_copy(h_hbm.at[idx(src_v, t)], rows_v.at[b], gsem)

      def gwait(t, b):
        pltpu.make_async_copy(h_hbm.at[idx(src_v, t)], rows_v.at[b],
                              gsem).wait()

      def scat(t, b):
        pltpu.async_copy(rows_v.at[b], acc_sh.at[idx(dst_v, t)], ssem,
                         add=True)

      def swait(t, b):
        pltpu.make_async_copy(rows_v.at[b], acc_sh.at[idx(dst_v, t)],
                              ssem).wait()

      gather(0, 0)

      def body(i, _):
        t0 = 2 * i
        t1 = t0 + 1
        gwait(t0, 0)
        @pl.when(i >= 1)
        def _():
          swait(t0 - 1, 1)
        gather(t1, 1)
        scat(t0, 0)
        gwait(t1, 1)
        swait(t0, 0)
        @pl.when(t1 + 1 < ns)
        def _():
          gather(t1 + 1, 0)
        scat(t1, 1)
        return 0
      lax.fori_loop(0, ns // 2, body, 0)
      swait(ns - 1, 1)
    plsc.subcore_barrier()

    pltpu.sync_copy(acc_sh.at[pl.ds(s * RPT, RPT)],
                    out_hbm.at[c, pl.ds(s * RPT, RPT)])

  return agg


_agg_hid = _make_agg(D_HID, 1, 2)
_agg_cls = _make_agg(DC, 4, 1)

_deg = functools.partial(
    pl.kernel,
    out_type=jax.ShapeDtypeStruct((NC, NPD), jnp.float32),
    mesh=plsc.VectorSubcoreMesh(**_MESH),
    compiler_params=pltpu.CompilerParams(use_tc_tiling_on_sc=False),
    scratch_types=[
        pltpu.VMEM((_DNS, _DL), jnp.int32),
        pltpu.VMEM((512,), jnp.float32),
        pltpu.VMEM((128,), jnp.float32),
        pltpu.VMEM_SHARED((NPD,), jnp.float32),
        pltpu.SemaphoreType.DMA,
    ],
)


_BM = 1024


def _mma_body(x_ref, w1_ref, h_ref):
  h_ref[...] = jnp.dot(x_ref[...], w1_ref[...],
                       preferred_element_type=jnp.float32)


def _mmb_body(degp_ref, h_ref, h1p_ref, dinv_ref):
  deg = degp_ref[0, :] + degp_ref[1, :] + 1.0
  dinv = lax.rsqrt(deg)
  h1p_ref[...] = h_ref[...] * dinv[:, None]
  dinv_ref[...] = dinv[:, None]


def _mm2_body(p_ref, dinv_ref, b1_ref, w2_ref, h2p_ref):
  dinv = dinv_ref[...]
  u = (p_ref[0] + p_ref[1]) * dinv + b1_ref[...]
  h = jnp.maximum(u, 0.0)
  h2p_ref[...] = jnp.dot(h, w2_ref[...],
                         preferred_element_type=jnp.float32) * dinv


def _fin_body(p_ref, dinv_ref, b2_ref, out_ref):
  out_ref[...] = ((p_ref[0] + p_ref[1]) * dinv_ref[...]
                  + b2_ref[...])[:, :N_CLS]


def _row_spec(bm, d):
  return pl.BlockSpec((bm, d), lambda i: (i, 0))


def _part_spec(bm, d):
  return pl.BlockSpec((NC, bm, d), lambda i: (0, i, 0))


def _full_spec(shape):
  return pl.BlockSpec(shape, lambda i: tuple(0 for _ in shape))


def kernel(x, edge_index, W1, b1, W2, b2):
  src = edge_index[0].reshape(TCH, CR)
  dst = edge_index[1].reshape(TCH, CR)
  xp = jnp.zeros((NPD, D_IN), jnp.float32).at[:N].set(x)
  w2p = jnp.zeros((D_HID, DC), jnp.float32).at[:, :N_CLS].set(W2)
  b2p = jnp.zeros((1, DC), jnp.float32).at[0, :N_CLS].set(b2)
  b1r = b1.reshape(1, D_HID)

  degp = _deg(_deg_kernel)(edge_index[1].reshape(E // _DL, _DL))

  grid = (NPD // _BM,)

  h1raw = pl.pallas_call(
      _mma_body,
      grid=grid,
      in_specs=[_row_spec(_BM, D_IN), _full_spec((D_IN, D_HID))],
      out_specs=_row_spec(_BM, D_HID),
      out_shape=jax.ShapeDtypeStruct((NPD, D_HID), jnp.float32),
  )(xp, W1)

  h1p, dinv = pl.pallas_call(
      _mmb_body,
      grid=grid,
      in_specs=[
          pl.BlockSpec((NC, _BM), lambda i: (0, i)),
          _row_spec(_BM, D_HID),
      ],
      out_specs=[_row_spec(_BM, D_HID), _row_spec(_BM, 1)],
      out_shape=[
          jax.ShapeDtypeStruct((NPD, D_HID), jnp.float32),
          jax.ShapeDtypeStruct((NPD, 1), jnp.float32),
      ],
  )(degp, h1raw)

  p1 = _agg_hid(h1p, src, dst)

  h2p = pl.pallas_call(
      _mm2_body,
      grid=grid,
      in_specs=[
          _part_spec(_BM, D_HID),
          _row_spec(_BM, 1),
          _full_spec((1, D_HID)),
          _full_spec((D_HID, DC)),
      ],
      out_specs=_row_spec(_BM, DC),
      out_shape=jax.ShapeDtypeStruct((NPD, DC), jnp.float32),
  )(p1, dinv, b1r, w2p)

  p2 = _agg_cls(h2p, src.reshape(E // 500, 500), dst.reshape(E // 500, 500))

  out = pl.pallas_call(
      _fin_body,
      grid=grid,
      in_specs=[
          _part_spec(_BM, DC),
          _row_spec(_BM, 1),
          _full_spec((1, DC)),
      ],
      out_specs=_row_spec(_BM, N_CLS),
      out_shape=jax.ShapeDtypeStruct((NPD, N_CLS), jnp.float32),
  )(p2, dinv, b2p)

  return out[:N]


# speedup vs baseline: 1.0328x; 1.0328x over previous
import functools

import jax
import jax.numpy as jnp
from jax import lax
from jax.experimental import pallas as pl
from jax.experimental.pallas import tpu as pltpu
import jax.experimental.pallas.tpu_sc as plsc

N = 10000
E = 320000
D_IN = 128
D_HID = 128
N_CLS = 40

NC, NS = 2, 16
NW = NC * NS
CR = 125
TCH = E // CR
WCH = TCH // NW
NPD = 10240
RPT = NPD // NS
DC = 48

_MESH = dict(core_axis_name="c", subcore_axis_name="s", num_cores=NC,
             num_subcores=NS)


_DL = 500
_DNS = E // (NW * _DL)


def _deg_kernel(dst_hbm, degp_hbm, dst_v, ones_v, zeros_v, deg_sh, sem):
  c = lax.axis_index("c")
  s = lax.axis_index("s")
  w = s * NC + c

  z16 = jnp.zeros((16,), jnp.float32)
  for i in range(512 // 16):
    ones_v[pl.ds(i * 16, 16)] = z16 + 1.0
  for i in range(128 // 16):
    zeros_v[pl.ds(i * 16, 16)] = z16
  for i in range(RPT // 128):
    pltpu.sync_copy(zeros_v, deg_sh.at[pl.ds(s * RPT + i * 128, 128)])
  plsc.subcore_barrier()

  pltpu.sync_copy(dst_hbm.at[pl.ds(w * _DNS, _DNS)], dst_v)
  def body(j, _):
    pltpu.sync_copy(ones_v.at[pl.ds(0, _DL)], deg_sh.at[dst_v.at[j]],
                    add=True)
    return 0
  lax.fori_loop(0, _DNS, body, 0)
  plsc.subcore_barrier()

  pltpu.sync_copy(deg_sh.at[pl.ds(s * RPT, RPT)],
                  degp_hbm.at[c, pl.ds(s * RPT, RPT)])


def _make_agg(d, multi, n_half):
  L = multi * CR
  ns = WCH // (n_half * multi)
  assert ns % 2 == 0

  rows_t = pltpu.VMEM((2, L, d), jnp.float32)

  @functools.partial(
      pl.kernel,
      out_type=jax.ShapeDtypeStruct((NC, NPD, d), jnp.float32),
      mesh=plsc.VectorSubcoreMesh(**_MESH),
      compiler_params=pltpu.CompilerParams(use_tc_tiling_on_sc=(d % 128 == 0)),
      scratch_types=[
          pltpu.VMEM((ns, L), jnp.int32),
          pltpu.VMEM((ns, L), jnp.int32),
          rows_t,
          pltpu.VMEM_SHARED((NPD, d), jnp.float32),
          pltpu.SemaphoreType.DMA,
          pltpu.SemaphoreType.DMA,
      ],
  )
  def agg(h_hbm, src_hbm, dst_hbm, out_hbm,
          src_v, dst_v, rows_v, acc_sh, gsem, ssem):
    c = lax.axis_index("c")
    s = lax.axis_index("s")
    w = s * NC + c

    @pl.when(c == 0)
    def _():
      pltpu.sync_copy(h_hbm.at[pl.ds(s * RPT, RPT)],
                      acc_sh.at[pl.ds(s * RPT, RPT)])

    @pl.when(c == 1)
    def _():
      z16 = jnp.zeros((16,), jnp.float32)
      def zb(i, _):
        for k in range(d // 16):
          rows_v[0, i, pl.ds(k * 16, 16)] = z16
        return 0
      lax.fori_loop(0, CR, zb, 0)
      for i in range(5):
        pltpu.sync_copy(rows_v.at[0, pl.ds(0, 120)],
                        acc_sh.at[pl.ds(s * RPT + i * 120, 120)])
      pltpu.sync_copy(rows_v.at[0, pl.ds(0, 40)],
                      acc_sh.at[pl.ds(s * RPT + 600, 40)])
    plsc.subcore_barrier()

    def idx(v, t):
      return v.at[t]

    for k in range(n_half):
      pltpu.sync_copy(src_hbm.at[pl.ds((w * n_half + k) * ns, ns)], src_v)
      pltpu.sync_copy(dst_hbm.at[pl.ds((w * n_half + k) * ns, ns)], dst_v)

      def gather(t, b):
        pltpu.async_copy(h_hbm.at[idx(src_v, t)], rows_v.at[b], gsem)

      def gwait(t, b):
        pltpu.make_async_copy(h_hbm.at[idx(src_v, t)], rows_v.at[b],
                              gsem).wait()

      def scat(t, b):
        pltpu.async_copy(rows_v.at[b], acc_sh.at[idx(dst_v, t)], ssem,
                         add=True)

      def swait(t, b):
        pltpu.make_async_copy(rows_v.at[b], acc_sh.at[idx(dst_v, t)],
                              ssem).wait()

      gather(0, 0)

      def body(i, _):
        t0 = 2 * i
        t1 = t0 + 1
        gwait(t0, 0)
        @pl.when(i >= 1)
        def _():
          swait(t0 - 1, 1)
        gather(t1, 1)
        scat(t0, 0)
        gwait(t1, 1)
        swait(t0, 0)
        @pl.when(t1 + 1 < ns)
        def _():
          gather(t1 + 1, 0)
        scat(t1, 1)
        return 0
      lax.fori_loop(0, ns // 2, body, 0)
      swait(ns - 1, 1)
    plsc.subcore_barrier()

    pltpu.sync_copy(acc_sh.at[pl.ds(s * RPT, RPT)],
                    out_hbm.at[c, pl.ds(s * RPT, RPT)])

  return agg


_agg_hid = _make_agg(D_HID, 1, 2)
_agg_cls = _make_agg(DC, 4, 1)

_deg = functools.partial(
    pl.kernel,
    out_type=jax.ShapeDtypeStruct((NC, NPD), jnp.float32),
    mesh=plsc.VectorSubcoreMesh(**_MESH),
    compiler_params=pltpu.CompilerParams(use_tc_tiling_on_sc=False),
    scratch_types=[
        pltpu.VMEM((_DNS, _DL), jnp.int32),
        pltpu.VMEM((512,), jnp.float32),
        pltpu.VMEM((128,), jnp.float32),
        pltpu.VMEM_SHARED((NPD,), jnp.float32),
        pltpu.SemaphoreType.DMA,
    ],
)


_BM = 1024


def _mm1_body(degp_ref, x_ref, w1_ref, h1p_ref, dinv_ref):
  deg = degp_ref[0, :] + degp_ref[1, :] + 1.0
  dinv = lax.rsqrt(deg)
  h = jnp.dot(x_ref[...], w1_ref[...], preferred_element_type=jnp.float32)
  h1p_ref[...] = h * dinv[:, None]
  dinv_ref[...] = dinv[:, None]


def _mm2_body(p_ref, dinv_ref, b1_ref, w2_ref, h2p_ref):
  dinv = dinv_ref[...]
  u = (p_ref[0] + p_ref[1]) * dinv + b1_ref[...]
  h = jnp.maximum(u, 0.0)
  h2p_ref[...] = jnp.dot(h, w2_ref[...],
                         preferred_element_type=jnp.float32) * dinv


def _fin_body(p_ref, dinv_ref, b2_ref, out_ref):
  out_ref[...] = ((p_ref[0] + p_ref[1]) * dinv_ref[...]
                  + b2_ref[...])[:, :N_CLS]


def _row_spec(bm, d):
  return pl.BlockSpec((bm, d), lambda i: (i, 0))


def _part_spec(bm, d):
  return pl.BlockSpec((NC, bm, d), lambda i: (0, i, 0))


def _full_spec(shape):
  return pl.BlockSpec(shape, lambda i: tuple(0 for _ in shape))


def kernel(x, edge_index, W1, b1, W2, b2):
  src = edge_index[0].reshape(TCH, CR)
  dst = edge_index[1].reshape(TCH, CR)
  xp = jnp.zeros((NPD, D_IN), jnp.float32).at[:N].set(x)
  w2p = jnp.zeros((D_HID, DC), jnp.float32).at[:, :N_CLS].set(W2)
  b2p = jnp.zeros((1, DC), jnp.float32).at[0, :N_CLS].set(b2)
  b1r = b1.reshape(1, D_HID)

  degp = _deg(_deg_kernel)(edge_index[1].reshape(E // _DL, _DL))

  grid = (NPD // _BM,)

  h1p, dinv = pl.pallas_call(
      _mm1_body,
      grid=grid,
      in_specs=[
          pl.BlockSpec((NC, _BM), lambda i: (0, i)),
          _row_spec(_BM, D_IN),
          _full_spec((D_IN, D_HID)),
      ],
      out_specs=[_row_spec(_BM, D_HID), _row_spec(_BM, 1)],
      out_shape=[
          jax.ShapeDtypeStruct((NPD, D_HID), jnp.float32),
          jax.ShapeDtypeStruct((NPD, 1), jnp.float32),
      ],
  )(degp, xp, W1)

  p1 = _agg_hid(h1p, src, dst)

  h2p = pl.pallas_call(
      _mm2_body,
      grid=grid,
      in_specs=[
          _part_spec(_BM, D_HID),
          _row_spec(_BM, 1),
          _full_spec((1, D_HID)),
          _full_spec((D_HID, DC)),
      ],
      out_specs=_row_spec(_BM, DC),
      out_shape=jax.ShapeDtypeStruct((NPD, DC), jnp.float32),
  )(p1, dinv, b1r, w2p)

  p2 = _agg_cls(h2p, src.reshape(E // 500, 500), dst.reshape(E // 500, 500))

  out = pl.pallas_call(
      _fin_body,
      grid=grid,
      in_specs=[
          _part_spec(_BM, DC),
          _row_spec(_BM, 1),
          _full_spec((1, DC)),
      ],
      out_specs=_row_spec(_BM, N_CLS),
      out_shape=jax.ShapeDtypeStruct((NPD, N_CLS), jnp.float32),
  )(p2, dinv, b2p)

  return out[:N]


# speedup vs baseline: 1.2223x; 1.1835x over previous
import functools

import jax
import jax.numpy as jnp
from jax import lax
from jax.experimental import pallas as pl
from jax.experimental.pallas import tpu as pltpu
import jax.experimental.pallas.tpu_sc as plsc

N = 10000
E = 320000
D_IN = 128
D_HID = 128
N_CLS = 40

NC, NS = 2, 16
NW = NC * NS
CR = 125
TCH = E // CR
WCH = TCH // NW
NPD = 10240
RPT = NPD // NS
DC = 48

_MESH = dict(core_axis_name="c", subcore_axis_name="s", num_cores=NC,
             num_subcores=NS)


_DL = 500
_DNS = E // (NW * _DL)


def _deg_kernel(dst_hbm, degp_hbm, dst_v, ones_v, zeros_v, deg_sh, sem):
  c = lax.axis_index("c")
  s = lax.axis_index("s")
  w = s * NC + c

  z16 = jnp.zeros((16,), jnp.float32)
  for i in range(512 // 16):
    ones_v[pl.ds(i * 16, 16)] = z16 + 1.0
  for i in range(128 // 16):
    zeros_v[pl.ds(i * 16, 16)] = z16
  for i in range(RPT // 128):
    pltpu.sync_copy(zeros_v, deg_sh.at[pl.ds(s * RPT + i * 128, 128)])
  plsc.subcore_barrier()

  pltpu.sync_copy(dst_hbm.at[pl.ds(w * _DNS, _DNS)], dst_v)
  def body(j, _):
    pltpu.sync_copy(ones_v.at[pl.ds(0, _DL)], deg_sh.at[dst_v.at[j]],
                    add=True)
    return 0
  lax.fori_loop(0, _DNS, body, 0)
  plsc.subcore_barrier()

  pltpu.sync_copy(deg_sh.at[pl.ds(s * RPT, RPT)],
                  degp_hbm.at[c, pl.ds(s * RPT, RPT)])


def _make_agg(d, multi, n_half, dtype=jnp.float32, tc_tiling=None):
  L = multi * CR
  ns = WCH // (n_half * multi)
  assert ns % 2 == 0
  if tc_tiling is None:
    tc_tiling = d % 128 == 0 and dtype == jnp.float32

  rows_t = pltpu.VMEM((2, L, d), dtype)

  @functools.partial(
      pl.kernel,
      out_type=jax.ShapeDtypeStruct((NC, NPD, d), dtype),
      mesh=plsc.VectorSubcoreMesh(**_MESH),
      compiler_params=pltpu.CompilerParams(use_tc_tiling_on_sc=tc_tiling),
      scratch_types=[
          pltpu.VMEM((ns, L), jnp.int32),
          pltpu.VMEM((ns, L), jnp.int32),
          rows_t,
          pltpu.VMEM_SHARED((NPD, d), dtype),
          pltpu.SemaphoreType.DMA,
          pltpu.SemaphoreType.DMA,
      ],
  )
  def agg(h_hbm, src_hbm, dst_hbm, out_hbm,
          src_v, dst_v, rows_v, acc_sh, gsem, ssem):
    c = lax.axis_index("c")
    s = lax.axis_index("s")
    w = s * NC + c

    @pl.when(c == 0)
    def _():
      pltpu.sync_copy(h_hbm.at[pl.ds(s * RPT, RPT)],
                      acc_sh.at[pl.ds(s * RPT, RPT)])

    @pl.when(c == 1)
    def _():
      zlanes = 16 if dtype == jnp.float32 else 32
      zv = jnp.zeros((zlanes,), dtype)
      def zb(i, _):
        for k in range(d // zlanes):
          rows_v[0, i, pl.ds(k * zlanes, zlanes)] = zv
        return 0
      lax.fori_loop(0, CR, zb, 0)
      for i in range(5):
        pltpu.sync_copy(rows_v.at[0, pl.ds(0, 120)],
                        acc_sh.at[pl.ds(s * RPT + i * 120, 120)])
      pltpu.sync_copy(rows_v.at[0, pl.ds(0, 40)],
                      acc_sh.at[pl.ds(s * RPT + 600, 40)])
    plsc.subcore_barrier()

    def idx(v, t):
      return v.at[t]

    for k in range(n_half):
      pltpu.sync_copy(src_hbm.at[pl.ds((w * n_half + k) * ns, ns)], src_v)
      pltpu.sync_copy(dst_hbm.at[pl.ds((w * n_half + k) * ns, ns)], dst_v)

      def gather(t, b):
        pltpu.async_copy(h_hbm.at[idx(src_v, t)], rows_v.at[b], gsem)

      def gwait(t, b):
        pltpu.make_async_copy(h_hbm.at[idx(src_v, t)], rows_v.at[b],
                              gsem).wait()

      def scat(t, b):
        pltpu.async_copy(rows_v.at[b], acc_sh.at[idx(dst_v, t)], ssem,
                         add=True)

      def swait(t, b):
        pltpu.make_async_copy(rows_v.at[b], acc_sh.at[idx(dst_v, t)],
                              ssem).wait()

      gather(0, 0)

      def body(i, _):
        t0 = 2 * i
        t1 = t0 + 1
        gwait(t0, 0)
        @pl.when(i >= 1)
        def _():
          swait(t0 - 1, 1)
        gather(t1, 1)
        scat(t0, 0)
        gwait(t1, 1)
        swait(t0, 0)
        @pl.when(t1 + 1 < ns)
        def _():
          gather(t1 + 1, 0)
        scat(t1, 1)
        return 0
      lax.fori_loop(0, ns // 2, body, 0)
      swait(ns - 1, 1)
    plsc.subcore_barrier()

    pltpu.sync_copy(acc_sh.at[pl.ds(s * RPT, RPT)],
                    out_hbm.at[c, pl.ds(s * RPT, RPT)])

  return agg


_agg_hid = _make_agg(D_HID, 4, 1, dtype=jnp.bfloat16)
_agg_cls = _make_agg(DC, 4, 1)

_deg = functools.partial(
    pl.kernel,
    out_type=jax.ShapeDtypeStruct((NC, NPD), jnp.float32),
    mesh=plsc.VectorSubcoreMesh(**_MESH),
    compiler_params=pltpu.CompilerParams(use_tc_tiling_on_sc=False),
    scratch_types=[
        pltpu.VMEM((_DNS, _DL), jnp.int32),
        pltpu.VMEM((512,), jnp.float32),
        pltpu.VMEM((128,), jnp.float32),
        pltpu.VMEM_SHARED((NPD,), jnp.float32),
        pltpu.SemaphoreType.DMA,
    ],
)


_BM = 1024


def _mm1_body(degp_ref, x_ref, w1_ref, h1p_ref, dinv_ref):
  deg = degp_ref[0, :] + degp_ref[1, :] + 1.0
  dinv = lax.rsqrt(deg)
  h = jnp.dot(x_ref[...], w1_ref[...], preferred_element_type=jnp.float32)
  h1p_ref[...] = (h * dinv[:, None]).astype(jnp.bfloat16)
  dinv_ref[...] = dinv[:, None]


def _mm2_body(p_ref, dinv_ref, b1_ref, w2_ref, h2p_ref):
  dinv = dinv_ref[...]
  psum = p_ref[0].astype(jnp.float32) + p_ref[1].astype(jnp.float32)
  u = psum * dinv + b1_ref[...]
  h = jnp.maximum(u, 0.0)
  h2p_ref[...] = jnp.dot(h, w2_ref[...],
                         preferred_element_type=jnp.float32) * dinv


def _fin_body(p_ref, dinv_ref, b2_ref, out_ref):
  out_ref[...] = ((p_ref[0] + p_ref[1]) * dinv_ref[...]
                  + b2_ref[...])[:, :N_CLS]


def _row_spec(bm, d):
  return pl.BlockSpec((bm, d), lambda i: (i, 0))


def _part_spec(bm, d):
  return pl.BlockSpec((NC, bm, d), lambda i: (0, i, 0))


def _full_spec(shape):
  return pl.BlockSpec(shape, lambda i: tuple(0 for _ in shape))


def kernel(x, edge_index, W1, b1, W2, b2):
  src = edge_index[0].reshape(TCH, CR)
  dst = edge_index[1].reshape(TCH, CR)
  xp = jnp.zeros((NPD, D_IN), jnp.float32).at[:N].set(x)
  w2p = jnp.zeros((D_HID, DC), jnp.float32).at[:, :N_CLS].set(W2)
  b2p = jnp.zeros((1, DC), jnp.float32).at[0, :N_CLS].set(b2)
  b1r = b1.reshape(1, D_HID)

  degp = _deg(_deg_kernel)(edge_index[1].reshape(E // _DL, _DL))

  grid = (NPD // _BM,)

  h1p, dinv = pl.pallas_call(
      _mm1_body,
      grid=grid,
      in_specs=[
          pl.BlockSpec((NC, _BM), lambda i: (0, i)),
          _row_spec(_BM, D_IN),
          _full_spec((D_IN, D_HID)),
      ],
      out_specs=[_row_spec(_BM, D_HID), _row_spec(_BM, 1)],
      out_shape=[
          jax.ShapeDtypeStruct((NPD, D_HID), jnp.bfloat16),
          jax.ShapeDtypeStruct((NPD, 1), jnp.float32),
      ],
  )(degp, xp, W1)

  p1 = _agg_hid(h1p, src.reshape(E // 500, 500), dst.reshape(E // 500, 500))

  h2p = pl.pallas_call(
      _mm2_body,
      grid=grid,
      in_specs=[
          _part_spec(_BM, D_HID),
          _row_spec(_BM, 1),
          _full_spec((1, D_HID)),
          _full_spec((D_HID, DC)),
      ],
      out_specs=_row_spec(_BM, DC),
      out_shape=jax.ShapeDtypeStruct((NPD, DC), jnp.float32),
  )(p1, dinv, b1r, w2p)

  p2 = _agg_cls(h2p, src.reshape(E // 500, 500), dst.reshape(E // 500, 500))

  out = pl.pallas_call(
      _fin_body,
      grid=grid,
      in_specs=[
          _part_spec(_BM, DC),
          _row_spec(_BM, 1),
          _full_spec((1, DC)),
      ],
      out_specs=_row_spec(_BM, N_CLS),
      out_shape=jax.ShapeDtypeStruct((NPD, N_CLS), jnp.float32),
  )(p2, dinv, b2p)

  return out[:N]


# speedup vs baseline: 1.2859x; 1.0520x over previous
import functools

import jax
import jax.numpy as jnp
from jax import lax
from jax.experimental import pallas as pl
from jax.experimental.pallas import tpu as pltpu
import jax.experimental.pallas.tpu_sc as plsc

N = 10000
E = 320000
D_IN = 128
D_HID = 128
N_CLS = 40

NC, NS = 2, 16
NW = NC * NS
CR = 125
TCH = E // CR
WCH = TCH // NW
NPD = 10240
RPT = NPD // NS
DC = 64

_MESH = dict(core_axis_name="c", subcore_axis_name="s", num_cores=NC,
             num_subcores=NS)


_DL = 1000
_DNS = E // (NW * _DL)


def _deg_kernel(dst_hbm, degp_hbm, dst_v, ones_v, zeros_v, deg_sh, sem):
  c = lax.axis_index("c")
  s = lax.axis_index("s")
  w = s * NC + c

  z16 = jnp.zeros((16,), jnp.float32)
  for i in range(1024 // 16):
    ones_v[pl.ds(i * 16, 16)] = z16 + 1.0
  for i in range(128 // 16):
    zeros_v[pl.ds(i * 16, 16)] = z16
  for i in range(RPT // 128):
    pltpu.sync_copy(zeros_v, deg_sh.at[pl.ds(s * RPT + i * 128, 128)])
  plsc.subcore_barrier()

  pltpu.sync_copy(dst_hbm.at[pl.ds(w * _DNS, _DNS)], dst_v)
  def body(j, _):
    pltpu.sync_copy(ones_v.at[pl.ds(0, _DL)], deg_sh.at[dst_v.at[j]],
                    add=True)
    return 0
  lax.fori_loop(0, _DNS, body, 0)
  plsc.subcore_barrier()

  pltpu.sync_copy(deg_sh.at[pl.ds(s * RPT, RPT)],
                  degp_hbm.at[c, pl.ds(s * RPT, RPT)])


def _make_agg(d, multi, n_half, dtype=jnp.float32, tc_tiling=None):
  L = multi * CR
  ns = WCH // (n_half * multi)
  assert ns % 2 == 0
  if tc_tiling is None:
    tc_tiling = d % 128 == 0 and dtype == jnp.float32

  rows_t = pltpu.VMEM((2, L, d), dtype)

  @functools.partial(
      pl.kernel,
      out_type=jax.ShapeDtypeStruct((NC, NPD, d), dtype),
      mesh=plsc.VectorSubcoreMesh(**_MESH),
      compiler_params=pltpu.CompilerParams(use_tc_tiling_on_sc=tc_tiling),
      scratch_types=[
          pltpu.VMEM((ns, L), jnp.int32),
          pltpu.VMEM((ns, L), jnp.int32),
          rows_t,
          pltpu.VMEM_SHARED((NPD, d), dtype),
          pltpu.SemaphoreType.DMA,
          pltpu.SemaphoreType.DMA,
      ],
  )
  def agg(h_hbm, src_hbm, dst_hbm, out_hbm,
          src_v, dst_v, rows_v, acc_sh, gsem, ssem):
    c = lax.axis_index("c")
    s = lax.axis_index("s")
    w = s * NC + c

    @pl.when(c == 0)
    def _():
      pltpu.sync_copy(h_hbm.at[pl.ds(s * RPT, RPT)],
                      acc_sh.at[pl.ds(s * RPT, RPT)])

    @pl.when(c == 1)
    def _():
      zlanes = 16 if dtype == jnp.float32 else 32
      zv = jnp.zeros((zlanes,), dtype)
      def zb(i, _):
        for k in range(d // zlanes):
          rows_v[0, i, pl.ds(k * zlanes, zlanes)] = zv
        return 0
      lax.fori_loop(0, CR, zb, 0)
      for i in range(5):
        pltpu.sync_copy(rows_v.at[0, pl.ds(0, 120)],
                        acc_sh.at[pl.ds(s * RPT + i * 120, 120)])
      pltpu.sync_copy(rows_v.at[0, pl.ds(0, 40)],
                      acc_sh.at[pl.ds(s * RPT + 600, 40)])
    plsc.subcore_barrier()

    def idx(v, t):
      return v.at[t]

    for k in range(n_half):
      pltpu.sync_copy(src_hbm.at[pl.ds((w * n_half + k) * ns, ns)], src_v)
      pltpu.sync_copy(dst_hbm.at[pl.ds((w * n_half + k) * ns, ns)], dst_v)

      def gather(t, b):
        pltpu.async_copy(h_hbm.at[idx(src_v, t)], rows_v.at[b], gsem)

      def gwait(t, b):
        pltpu.make_async_copy(h_hbm.at[idx(src_v, t)], rows_v.at[b],
                              gsem).wait()

      def scat(t, b):
        pltpu.async_copy(rows_v.at[b], acc_sh.at[idx(dst_v, t)], ssem,
                         add=True)

      def swait(t, b):
        pltpu.make_async_copy(rows_v.at[b], acc_sh.at[idx(dst_v, t)],
                              ssem).wait()

      gather(0, 0)

      def body(i, _):
        t0 = 2 * i
        t1 = t0 + 1
        gwait(t0, 0)
        @pl.when(i >= 1)
        def _():
          swait(t0 - 1, 1)
        gather(t1, 1)
        scat(t0, 0)
        gwait(t1, 1)
        swait(t0, 0)
        @pl.when(t1 + 1 < ns)
        def _():
          gather(t1 + 1, 0)
        scat(t1, 1)
        return 0
      lax.fori_loop(0, ns // 2, body, 0)
      swait(ns - 1, 1)
    plsc.subcore_barrier()

    pltpu.sync_copy(acc_sh.at[pl.ds(s * RPT, RPT)],
                    out_hbm.at[c, pl.ds(s * RPT, RPT)])

  return agg


_agg_hid = _make_agg(D_HID, 4, 1, dtype=jnp.bfloat16)
_agg_cls = _make_agg(DC, 8, 1, dtype=jnp.bfloat16)

_deg = functools.partial(
    pl.kernel,
    out_type=jax.ShapeDtypeStruct((NC, NPD), jnp.float32),
    mesh=plsc.VectorSubcoreMesh(**_MESH),
    compiler_params=pltpu.CompilerParams(use_tc_tiling_on_sc=False),
    scratch_types=[
        pltpu.VMEM((_DNS, _DL), jnp.int32),
        pltpu.VMEM((1024,), jnp.float32),
        pltpu.VMEM((128,), jnp.float32),
        pltpu.VMEM_SHARED((NPD,), jnp.float32),
        pltpu.SemaphoreType.DMA,
    ],
)


_BM = 1024


def _mm1_body(degp_ref, x_ref, w1_ref, h1p_ref, dinv_ref):
  deg = degp_ref[0, :] + degp_ref[1, :] + 1.0
  dinv = lax.rsqrt(deg)
  h = jnp.dot(x_ref[...], w1_ref[...], preferred_element_type=jnp.float32)
  h1p_ref[...] = (h * dinv[:, None]).astype(jnp.bfloat16)
  dinv_ref[...] = dinv[:, None]


def _mm2_body(p_ref, dinv_ref, b1_ref, w2_ref, h2p_ref):
  dinv = dinv_ref[...]
  psum = p_ref[0].astype(jnp.float32) + p_ref[1].astype(jnp.float32)
  u = psum * dinv + b1_ref[...]
  h = jnp.maximum(u, 0.0)
  h2p_ref[...] = (jnp.dot(h, w2_ref[...],
                          preferred_element_type=jnp.float32)
                  * dinv).astype(jnp.bfloat16)


def _fin_body(p_ref, dinv_ref, b2_ref, out_ref):
  psum = p_ref[0].astype(jnp.float32) + p_ref[1].astype(jnp.float32)
  out_ref[...] = (psum * dinv_ref[...] + b2_ref[...])[:, :N_CLS]


def _row_spec(bm, d):
  return pl.BlockSpec((bm, d), lambda i: (i, 0))


def _part_spec(bm, d):
  return pl.BlockSpec((NC, bm, d), lambda i: (0, i, 0))


def _full_spec(shape):
  return pl.BlockSpec(shape, lambda i: tuple(0 for _ in shape))


def kernel(x, edge_index, W1, b1, W2, b2):
  src = edge_index[0].reshape(TCH, CR)
  dst = edge_index[1].reshape(TCH, CR)
  xp = jnp.zeros((NPD, D_IN), jnp.float32).at[:N].set(x)
  w2p = jnp.zeros((D_HID, DC), jnp.float32).at[:, :N_CLS].set(W2)
  b2p = jnp.zeros((1, DC), jnp.float32).at[0, :N_CLS].set(b2)
  b1r = b1.reshape(1, D_HID)

  degp = _deg(_deg_kernel)(edge_index[1].reshape(E // _DL, _DL))

  grid = (NPD // _BM,)

  h1p, dinv = pl.pallas_call(
      _mm1_body,
      grid=grid,
      in_specs=[
          pl.BlockSpec((NC, _BM), lambda i: (0, i)),
          _row_spec(_BM, D_IN),
          _full_spec((D_IN, D_HID)),
      ],
      out_specs=[_row_spec(_BM, D_HID), _row_spec(_BM, 1)],
      out_shape=[
          jax.ShapeDtypeStruct((NPD, D_HID), jnp.bfloat16),
          jax.ShapeDtypeStruct((NPD, 1), jnp.float32),
      ],
  )(degp, xp, W1)

  p1 = _agg_hid(h1p, src.reshape(E // 500, 500), dst.reshape(E // 500, 500))

  h2p = pl.pallas_call(
      _mm2_body,
      grid=grid,
      in_specs=[
          _part_spec(_BM, D_HID),
          _row_spec(_BM, 1),
          _full_spec((1, D_HID)),
          _full_spec((D_HID, DC)),
      ],
      out_specs=_row_spec(_BM, DC),
      out_shape=jax.ShapeDtypeStruct((NPD, DC), jnp.bfloat16),
  )(p1, dinv, b1r, w2p)

  p2 = _agg_cls(h2p, src.reshape(E // 1000, 1000),
                dst.reshape(E // 1000, 1000))

  out = pl.pallas_call(
      _fin_body,
      grid=grid,
      in_specs=[
          _part_spec(_BM, DC),
          _row_spec(_BM, 1),
          _full_spec((1, DC)),
      ],
      out_specs=_row_spec(_BM, N_CLS),
      out_shape=jax.ShapeDtypeStruct((NPD, N_CLS), jnp.float32),
  )(p2, dinv, b2p)

  return out[:N]
